# baseline (speedup 1.0000x reference)
import functools

import jax
import jax.numpy as jnp
from jax import lax
from jax.experimental import pallas as pl
from jax.experimental.pallas import tpu as pltpu
from jax.experimental.pallas import tpu_sc as plsc

VOCAB = 1000000
EMB = 64
BATCH = 4096
HIST = 200

NC = 2
NS = 16
NW = NC * NS
TOTAL = BATCH * HIST
PER_W = TOTAL // NW
CHUNK = 128
NCHUNK = PER_W // CHUNK
NBUF = 4


def _emb_kernel(x_hbm, w_hbm, emb_hbm, mask_hbm,
                idx_vm, mask_vm, r0, r1, r2, r3, s0, s1, s2, s3):
    rbufs = (r0, r1, r2, r3)
    sems = (s0, s1, s2, s3)
    wid = lax.axis_index("s") * NC + lax.axis_index("c")
    base = wid * PER_W

    pltpu.sync_copy(x_hbm.at[wid], idx_vm)

    def gather(jj, b):
        return pltpu.make_async_copy(
            w_hbm.at[idx_vm.at[jj]], rbufs[b], sems[b])

    for b in range(NBUF):
        gather(b, b).start()

    def mask_row(j, carry):
        for k in range(CHUNK // 16):
            v = idx_vm[j, pl.ds(16 * k, 16)]
            mask_vm[j, pl.ds(16 * k, 16)] = jnp.where(
                v != 0, jnp.float32(1.0), jnp.float32(0.0))
        return carry
    lax.fori_loop(0, NCHUNK, mask_row, 0)
    pltpu.sync_copy(mask_vm, mask_hbm.at[wid])

    def main_body(g, carry):
        for b in range(NBUF):
            jj = g * NBUF + b
            gather(jj, b).wait()
            pltpu.sync_copy(rbufs[b], emb_hbm.at[pl.ds(base + jj * CHUNK, CHUNK)])
            gather(jj + NBUF, b).start()
        return carry
    lax.fori_loop(0, NCHUNK // NBUF - 1, main_body, 0)

    for b in range(NBUF):
        jj = NCHUNK - NBUF + b
        gather(jj, b).wait()
        pltpu.sync_copy(rbufs[b], emb_hbm.at[pl.ds(base + jj * CHUNK, CHUNK)])


@functools.partial(jax.jit, static_argnums=())
def kernel(x, W):
    x_r = x.astype(jnp.int32).reshape(NW, NCHUNK, CHUNK)
    kfn = functools.partial(
        pl.kernel,
        out_type=[
            jax.ShapeDtypeStruct((TOTAL, EMB), jnp.float32),
            jax.ShapeDtypeStruct((NW, NCHUNK, CHUNK), jnp.float32),
        ],
        mesh=plsc.VectorSubcoreMesh(core_axis_name="c", subcore_axis_name="s"),
        compiler_params=pltpu.CompilerParams(use_tc_tiling_on_sc=False),
        scratch_types=[
            pltpu.VMEM((NCHUNK, CHUNK), jnp.int32),
            pltpu.VMEM((NCHUNK, CHUNK), jnp.float32),
            pltpu.VMEM((CHUNK, EMB), jnp.float32),
            pltpu.VMEM((CHUNK, EMB), jnp.float32),
            pltpu.VMEM((CHUNK, EMB), jnp.float32),
            pltpu.VMEM((CHUNK, EMB), jnp.float32),
            pltpu.SemaphoreType.DMA,
            pltpu.SemaphoreType.DMA,
            pltpu.SemaphoreType.DMA,
            pltpu.SemaphoreType.DMA,
        ],
    )(_emb_kernel)
    emb, mask = kfn(x_r, W)
    return (emb.reshape(BATCH, HIST, EMB), mask.reshape(BATCH, HIST))
